# baseline (speedup 1.0000x reference)
import functools

import jax
import jax.numpy as jnp
from jax.experimental import pallas as pl
from jax.experimental.pallas import tpu as pltpu

_BLOCK_M = 400


def _gcn_body(x_ref, adj_ref, w1_ref, b1_ref, w2_ref, b2_ref, wr_ref, br_ref,
              out_ref, y1_ref, y2_ref, acc_ref, *, n_blocks, block_m, n_rows):
    p = pl.program_id(0)
    i = pl.program_id(1)

    @pl.when((p == 0) & (i == 0))
    def _init():
        y1_ref[...] = jnp.dot(x_ref[...], w1_ref[...],
                              precision=jax.lax.Precision.DEFAULT,
                              preferred_element_type=jnp.float32)
        acc_ref[...] = jnp.zeros_like(acc_ref)

    @pl.when(p == 0)
    def _layer1():
        s = jnp.dot(adj_ref[...], y1_ref[...],
                    precision=jax.lax.Precision.DEFAULT,
                    preferred_element_type=jnp.float32)
        h = jnp.maximum(s + b1_ref[...], 0.0)
        y2_ref[pl.ds(i * block_m, block_m), :] = jnp.dot(
            h, w2_ref[...],
            precision=jax.lax.Precision.DEFAULT,
            preferred_element_type=jnp.float32)

    @pl.when(p == 1)
    def _layer2():
        t = jnp.dot(adj_ref[...], y2_ref[...],
                    precision=jax.lax.Precision.DEFAULT,
                    preferred_element_type=jnp.float32)
        r = jnp.maximum(t + b2_ref[...], 0.0)
        acc_ref[...] += jnp.sum(r, axis=0, keepdims=True)

    @pl.when((p == 1) & (i == n_blocks - 1))
    def _readout():
        g = acc_ref[...] * (1.0 / n_rows)
        out_ref[...] = jnp.dot(g, wr_ref[...],
                               precision=jax.lax.Precision.DEFAULT,
                               preferred_element_type=jnp.float32) + br_ref[...]


def kernel(x, adj, W1, b1, W2, b2, Wr, br):
    n, f = x.shape
    h = W1.shape[1]
    op = Wr.shape[1]
    block_m = _BLOCK_M if n % _BLOCK_M == 0 else 8
    n_blocks = n // block_m

    out = pl.pallas_call(
        functools.partial(_gcn_body, n_blocks=n_blocks, block_m=block_m,
                          n_rows=n),
        grid=(2, n_blocks),
        in_specs=[
            pl.BlockSpec((n, f), lambda p, i: (0, 0)),
            pl.BlockSpec(
                (block_m, n),
                lambda p, i: (jnp.where(p == 0, i, n_blocks - 1 - i), 0)),
            pl.BlockSpec((f, h), lambda p, i: (0, 0)),
            pl.BlockSpec((1, h), lambda p, i: (0, 0)),
            pl.BlockSpec((h, h), lambda p, i: (0, 0)),
            pl.BlockSpec((1, h), lambda p, i: (0, 0)),
            pl.BlockSpec((h, op), lambda p, i: (0, 0)),
            pl.BlockSpec((1, op), lambda p, i: (0, 0)),
        ],
        out_specs=pl.BlockSpec((1, op), lambda p, i: (0, 0)),
        out_shape=jax.ShapeDtypeStruct((1, op), jnp.float32),
        scratch_shapes=[
            pltpu.VMEM((n, h), jnp.float32),
            pltpu.VMEM((n, h), jnp.float32),
            pltpu.VMEM((1, h), jnp.float32),
        ],
    )(x, adj, W1, b1.reshape(1, h), W2, b2.reshape(1, h), Wr,
      br.reshape(1, op))
    return out.reshape(op // 4, 4)


# speedup vs baseline: 1.0019x; 1.0019x over previous
import functools

import jax
import jax.numpy as jnp
from jax.experimental import pallas as pl
from jax.experimental.pallas import tpu as pltpu

_BLOCK_M = 400


def _gcn_body(x_ref, adj_ref, w1_ref, b1_ref, w2_ref, b2_ref, wr_ref, br_ref,
              out_ref, y1_ref, y2_ref, acc_ref, *, n_blocks, block_m, n_rows):
    p = pl.program_id(0)
    i = pl.program_id(1)

    @pl.when((p == 0) & (i == 0))
    def _init():
        y1_ref[...] = jnp.dot(x_ref[...], w1_ref[...],
                              precision=jax.lax.Precision.DEFAULT,
                              preferred_element_type=jnp.float32)
        acc_ref[...] = jnp.zeros_like(acc_ref)

    @pl.when(p == 0)
    def _layer1():
        s = jnp.dot(adj_ref[...], y1_ref[...],
                    precision=jax.lax.Precision.DEFAULT,
                    preferred_element_type=jnp.float32)
        h = jnp.maximum(s + b1_ref[...], 0.0)
        y2_ref[pl.ds(i * block_m, block_m), :] = jnp.dot(
            h, w2_ref[...],
            precision=jax.lax.Precision.DEFAULT,
            preferred_element_type=jnp.float32)

    @pl.when(p == 1)
    def _layer2():
        t = jnp.dot(adj_ref[...], y2_ref[...],
                    precision=jax.lax.Precision.DEFAULT,
                    preferred_element_type=jnp.float32)
        r = jnp.maximum(t + b2_ref[...], 0.0)
        acc_ref[...] += jnp.sum(r, axis=0, keepdims=True)

    @pl.when((p == 1) & (i == n_blocks - 1))
    def _readout():
        g = acc_ref[...] * (1.0 / n_rows)
        out_ref[...] = jnp.dot(g, wr_ref[...],
                               precision=jax.lax.Precision.DEFAULT,
                               preferred_element_type=jnp.float32) + br_ref[...]


def kernel(x, adj, W1, b1, W2, b2, Wr, br):
    n, f = x.shape
    h = W1.shape[1]
    op = Wr.shape[1]
    block_m = _BLOCK_M if n % _BLOCK_M == 0 else 8
    n_blocks = n // block_m

    out = pl.pallas_call(
        functools.partial(_gcn_body, n_blocks=n_blocks, block_m=block_m,
                          n_rows=n),
        grid=(2, n_blocks),
        in_specs=[
            pl.BlockSpec((n, f), lambda p, i: (0, 0)),
            pl.BlockSpec((block_m, n), lambda p, i: (i, 0)),
            pl.BlockSpec((f, h), lambda p, i: (0, 0)),
            pl.BlockSpec((1, h), lambda p, i: (0, 0)),
            pl.BlockSpec((h, h), lambda p, i: (0, 0)),
            pl.BlockSpec((1, h), lambda p, i: (0, 0)),
            pl.BlockSpec((h, op), lambda p, i: (0, 0)),
            pl.BlockSpec((1, op), lambda p, i: (0, 0)),
        ],
        out_specs=pl.BlockSpec((1, op), lambda p, i: (0, 0)),
        out_shape=jax.ShapeDtypeStruct((1, op), jnp.float32),
        scratch_shapes=[
            pltpu.VMEM((n, h), jnp.float32),
            pltpu.VMEM((n, h), jnp.float32),
            pltpu.VMEM((1, h), jnp.float32),
        ],
    )(x, adj, W1, b1.reshape(1, h), W2, b2.reshape(1, h), Wr,
      br.reshape(1, op))
    return out.reshape(op // 4, 4)


# speedup vs baseline: 1.0036x; 1.0018x over previous
import functools

import jax
import jax.numpy as jnp
from jax.experimental import pallas as pl
from jax.experimental.pallas import tpu as pltpu

_BLOCK_M = 400


def _gcn_body(x_ref, adj_ref, w1_ref, b1_ref, w2_ref, b2_ref, wr_ref, br_ref,
              out_ref, y1_ref, y2_ref, acc_ref, *, n_blocks, block_m, n_rows):
    p = pl.program_id(0)
    i = pl.program_id(1)

    @pl.when((p == 0) & (i == 0))
    def _init():
        for c in range(n_blocks):
            y1_ref[c * block_m:(c + 1) * block_m, :] = jnp.dot(
                x_ref[c * block_m:(c + 1) * block_m, :], w1_ref[...],
                precision=jax.lax.Precision.DEFAULT,
                preferred_element_type=jnp.float32)
        acc_ref[...] = jnp.zeros_like(acc_ref)

    @pl.when(p == 0)
    def _layer1():
        s = jnp.dot(adj_ref[...], y1_ref[...],
                    precision=jax.lax.Precision.DEFAULT,
                    preferred_element_type=jnp.float32)
        h = jnp.maximum(s + b1_ref[...], 0.0)
        y2_ref[pl.ds(i * block_m, block_m), :] = jnp.dot(
            h, w2_ref[...],
            precision=jax.lax.Precision.DEFAULT,
            preferred_element_type=jnp.float32)

    @pl.when(p == 1)
    def _layer2():
        t = jnp.dot(adj_ref[...], y2_ref[...],
                    precision=jax.lax.Precision.DEFAULT,
                    preferred_element_type=jnp.float32)
        r = jnp.maximum(t + b2_ref[...], 0.0)
        acc_ref[...] += jnp.sum(r, axis=0, keepdims=True)

    @pl.when((p == 1) & (i == n_blocks - 1))
    def _readout():
        g = acc_ref[...] * (1.0 / n_rows)
        out_ref[...] = jnp.dot(g, wr_ref[...],
                               precision=jax.lax.Precision.DEFAULT,
                               preferred_element_type=jnp.float32) + br_ref[...]


def kernel(x, adj, W1, b1, W2, b2, Wr, br):
    n, f = x.shape
    h = W1.shape[1]
    op = Wr.shape[1]
    block_m = _BLOCK_M if n % _BLOCK_M == 0 else 8
    n_blocks = n // block_m

    out = pl.pallas_call(
        functools.partial(_gcn_body, n_blocks=n_blocks, block_m=block_m,
                          n_rows=n),
        grid=(2, n_blocks),
        in_specs=[
            pl.BlockSpec((n, f), lambda p, i: (0, 0)),
            pl.BlockSpec((block_m, n), lambda p, i: (i, 0)),
            pl.BlockSpec((f, h), lambda p, i: (0, 0)),
            pl.BlockSpec((1, h), lambda p, i: (0, 0)),
            pl.BlockSpec((h, h), lambda p, i: (0, 0)),
            pl.BlockSpec((1, h), lambda p, i: (0, 0)),
            pl.BlockSpec((h, op), lambda p, i: (0, 0)),
            pl.BlockSpec((1, op), lambda p, i: (0, 0)),
        ],
        out_specs=pl.BlockSpec((1, op), lambda p, i: (0, 0)),
        out_shape=jax.ShapeDtypeStruct((1, op), jnp.float32),
        scratch_shapes=[
            pltpu.VMEM((n, h), jnp.float32),
            pltpu.VMEM((n, h), jnp.float32),
            pltpu.VMEM((1, h), jnp.float32),
        ],
    )(x, adj, W1, b1.reshape(1, h), W2, b2.reshape(1, h), Wr,
      br.reshape(1, op))
    return out.reshape(op // 4, 4)
